# baseline (speedup 1.0000x reference)
import functools
import jax
import jax.numpy as jnp
from jax import lax
from jax.experimental import pallas as pl
from jax.experimental.pallas import tpu as pltpu
from jax.experimental.pallas import tpu_sc as plsc

D = 128
N_ENT = 50000
BLK = 2000
NBLK = N_ENT // BLK

NSUB = 16
LANES = 16
NNZ_PAD = 204800
SLICE = NNZ_PAD // NSUB
EB = 1280
NBLK_E = SLICE // EB
NSTEP_E = EB // LANES
NE_PAD = 65536
CCH = 8192
NCHUNK_SC = 4
ROWS_PT = CCH // NSUB
TROWS = 1024
DUMMY_REL = 1012
DUMMY_ENT = NE_PAD - 1


_NFULL = N_ENT // D
_TAIL = N_ENT - _NFULL * D


def _ka_body(vt_ref, x_ref, out_ref):
    def step(j, acc):
        a = vt_ref[:, pl.ds(j * D, D)]
        b = x_ref[pl.ds(j * D, D), :]
        return acc + jnp.dot(a, b, preferred_element_type=jnp.float32)

    acc = jax.lax.fori_loop(0, _NFULL, step,
                            jnp.zeros((D, D), jnp.float32))
    a = vt_ref[:, pl.ds(_NFULL * D, _TAIL)]
    b = x_ref[pl.ds(_NFULL * D, _TAIL), :]
    out_ref[...] = acc + jnp.dot(a, b, preferred_element_type=jnp.float32)


def _mm_vt(vt, x):
    return pl.pallas_call(
        _ka_body,
        grid=(1,),
        in_specs=[
            pl.BlockSpec((D, N_ENT), lambda k: (0, 0)),
            pl.BlockSpec((N_ENT, D), lambda k: (0, 0)),
        ],
        out_specs=pl.BlockSpec((D, D), lambda k: (0, 0)),
        out_shape=jax.ShapeDtypeStruct((D, D), jnp.float32),
    )(vt, x)


def _stage_a(vt, e0, u):
    return _mm_vt(vt, e0), _mm_vt(vt, u)


def _kb_body(y1_ref, s_ref, wp_ref, b_ref, e0_ref, u_ref, agg_ref, rinv_ref,
             out_ref):
    y1 = y1_ref[...]
    y2 = y1 + jnp.dot(s_ref[...], y1, preferred_element_type=jnp.float32)
    ysum = y1 + y2
    e = e0_ref[...] + jnp.dot(u_ref[...], ysum,
                              preferred_element_type=jnp.float32)
    rel = agg_ref[0] * rinv_ref[0]
    acc = jnp.dot(e, wp_ref[0:D, :], preferred_element_type=jnp.float32)
    acc += jnp.dot(rel, wp_ref[D:2 * D, :], preferred_element_type=jnp.float32)
    out_ref[...] = jnp.maximum(acc + b_ref[...], 0.0)


def _stage_b(y1, s, w_proj, b2d, e0, u, sd, agg2, rinv2):
    return pl.pallas_call(
        _kb_body,
        grid=(NBLK,),
        in_specs=[
            pl.BlockSpec((D, D), lambda k: (0, 0)),
            pl.BlockSpec((D, D), lambda k: (0, 0)),
            pl.BlockSpec((2 * D, D), lambda k: (0, 0)),
            pl.BlockSpec((1, D), lambda k: (0, 0)),
            pl.BlockSpec((BLK, D), lambda k: (k, 0)),
            pl.BlockSpec((BLK, D), lambda k: (k, 0)),
            pl.BlockSpec((1, BLK, D), lambda k, sd=sd: (sd, k, 0)),
            pl.BlockSpec((1, BLK, D), lambda k, sd=sd: (sd, k, 0)),
        ],
        out_specs=pl.BlockSpec((BLK, D), lambda k: (k, 0)),
        out_shape=jax.ShapeDtypeStruct((N_ENT, D), jnp.float32),
        compiler_params=pltpu.CompilerParams(
            dimension_semantics=("arbitrary",),
        ),
    )(y1, s, w_proj, b2d, e0, u, agg2, rinv2)


def _sc_body(table_h, ent_h, rel_h, zeros_h, agg_h, rinv_h, ebuf, rbuf,
             lbuf, rbufc, ring, zb, cgrid, cexp1, idx64, acc, acc_c,
             sems, sems2):
    c = lax.axis_index("c")
    s = lax.axis_index("s")
    iota16 = lax.iota(jnp.int32, LANES)
    z16f = jnp.zeros((LANES,), jnp.float32)
    ones16 = jnp.ones((LANES,), jnp.float32)
    pltpu.sync_copy(zeros_h, zb)
    for q in range(4):
        idx64[pl.ds(q * LANES, LANES)] = iota16 + q * LANES

    def side_body(sd, _):
        def fire_edges(j):
            off = s * SLICE + j * EB
            buf = j % 2
            pltpu.async_copy(ent_h.at[sd, pl.ds(off, EB)], ebuf.at[buf],
                             sems2.at[buf])
            pltpu.async_copy(rel_h.at[sd, pl.ds(off, EB)], rbuf.at[buf],
                             sems2.at[buf])

        def wait_edges(j):
            off = s * SLICE + j * EB
            buf = j % 2
            pltpu.make_async_copy(ent_h.at[sd, pl.ds(off, EB)], ebuf.at[buf],
                                  sems2.at[buf]).wait()
            pltpu.make_async_copy(rel_h.at[sd, pl.ds(off, EB)], rbuf.at[buf],
                                  sems2.at[buf]).wait()

        def fire_gather(k):
            pltpu.async_copy(table_h.at[sd].at[rbufc.at[k & 3]],
                             ring.at[k & 1], sems.at[k & 1])

        def scatter_batch(k):
            pltpu.make_async_copy(table_h.at[sd].at[rbufc.at[k & 3]],
                                  ring.at[k & 1], sems.at[k & 1]).wait()
            pltpu.sync_copy(ring.at[k & 1], acc.at[lbuf.at[k & 3]], add=True)

        def chunk_body(p, _):
            lo = (2 * p + c) * CCH

            def zacc(q, _):
                pltpu.sync_copy(zb, acc.at[pl.ds(s * ROWS_PT + q * 32, 32)])
                return 0

            lax.fori_loop(0, ROWS_PT // 32, zacc, 0)
            pltpu.sync_copy(zb.at[pl.ds(0, 4)], acc_c.at[pl.ds(4 * s, 4)])

            def zg(r, _):
                for q in range(8):
                    cgrid[r, pl.ds(q * LANES, LANES)] = z16f
                return 0

            lax.fori_loop(0, 64, zg, 0)
            plsc.subcore_barrier()

            fire_edges(0)

            def blk(j, cur):
                wait_edges(j)

                @pl.when(j + 1 < NBLK_E)
                def _():
                    fire_edges(j + 1)

                def step(i, cur):
                    e = ebuf[j % 2, pl.ds(i * LANES, LANES)]
                    r = rbuf[j % 2, pl.ds(i * LANES, LANES)]
                    m = (e >= lo) & (e < lo + CCH)
                    mi = m.astype(jnp.int32)
                    el = e - lo
                    incl = plsc.cumsum(mi)
                    pos = (incl - mi) + cur
                    plsc.store_scatter(lbuf, [(pos >> 7) & 3, pos & 127],
                                       el, mask=m)
                    plsc.store_scatter(rbufc, [(pos >> 7) & 3, pos & 127],
                                       r, mask=m)
                    plsc.addupdate_scatter(cgrid, [el >> 7, el & 127],
                                           ones16, mask=m)
                    new = cur + incl[15]
                    kx = new >> 7

                    @pl.when(kx > (cur >> 7))
                    def _():
                        @pl.when(kx >= 2)
                        def _():
                            scatter_batch(kx - 2)

                        fire_gather(kx - 1)

                    return new

                return lax.fori_loop(0, NSTEP_E, step, cur)

            cur = lax.fori_loop(0, NBLK_E, blk, jnp.int32(0))

            for q in range(8):
                pos = iota16 + (cur + q * LANES)
                plsc.store_scatter(lbuf, [(pos >> 7) & 3, pos & 127],
                                   jnp.zeros((LANES,), jnp.int32))
                plsc.store_scatter(rbufc, [(pos >> 7) & 3, pos & 127],
                                   jnp.full((LANES,), DUMMY_REL, jnp.int32))
            kf = cur >> 7

            @pl.when(kf >= 1)
            def _():
                scatter_batch(kf - 1)

            fire_gather(kf)
            scatter_batch(kf)
            pltpu.sync_copy(cgrid, acc_c.at[idx64], add=True)
            plsc.subcore_barrier()

            pltpu.sync_copy(acc.at[pl.ds(s * ROWS_PT, ROWS_PT)],
                            agg_h.at[sd, pl.ds(lo + s * ROWS_PT, ROWS_PT)])
            pltpu.sync_copy(acc_c, cgrid)

            def grp(g, _):
                def ex(q, _):
                    el0 = s * ROWS_PT + g * 32 + q * LANES
                    cv = cgrid[el0 >> 7, pl.ds(el0 & 127, LANES)]
                    inv = 1.0 / (cv + 1e-05)
                    for l in range(LANES):
                        v = jnp.full((LANES,), inv[l], jnp.float32)
                        for k in range(8):
                            cexp1[pl.ds((q * LANES + l) * D + k * LANES,
                                        LANES)] = v
                    return 0

                lax.fori_loop(0, 2, ex, 0)
                base = (lo + s * ROWS_PT + g * 32) * D
                pltpu.sync_copy(cexp1.at[pl.ds(0, 32 * D)],
                                rinv_h.at[sd, pl.ds(base, 32 * D)])
                return 0

            lax.fori_loop(0, ROWS_PT // 32, grp, 0)
            plsc.subcore_barrier()
            return 0

        lax.fori_loop(0, NCHUNK_SC, chunk_body, 0)
        return 0

    lax.fori_loop(0, 2, side_body, 0)


def _sc_agg(table2, ent2, rel2, zeros32):
    run = functools.partial(
        pl.kernel,
        out_type=[
            jax.ShapeDtypeStruct((2, NE_PAD, D), jnp.float32),
            jax.ShapeDtypeStruct((2, NE_PAD * D), jnp.float32),
        ],
        mesh=plsc.VectorSubcoreMesh(core_axis_name="c", subcore_axis_name="s"),
        compiler_params=pltpu.CompilerParams(needs_layout_passes=False),
        scratch_types=[
            pltpu.VMEM((2, EB), jnp.int32),
            pltpu.VMEM((2, EB), jnp.int32),
            pltpu.VMEM((4, 128), jnp.int32),
            pltpu.VMEM((4, 128), jnp.int32),
            pltpu.VMEM((2, 128, D), jnp.float32),
            pltpu.VMEM((32, D), jnp.float32),
            pltpu.VMEM((64, 128), jnp.float32),
            pltpu.VMEM((32 * D,), jnp.float32),
            pltpu.VMEM((64,), jnp.int32),
            pltpu.VMEM_SHARED((CCH, D), jnp.float32),
            pltpu.VMEM_SHARED((64, 128), jnp.float32),
            pltpu.SemaphoreType.DMA((2,)),
            pltpu.SemaphoreType.DMA((2,)),
        ],
    )(_sc_body)
    return run(table2, ent2, rel2, zeros32)


def _aug_table(rel_emb):
    return jnp.zeros((TROWS, D), jnp.float32).at[:rel_emb.shape[0]].set(rel_emb)


def _pad_edges(ent_idx, rel_idx):
    npad = NNZ_PAD - ent_idx.shape[0]
    ent = jnp.concatenate([ent_idx, jnp.full((npad,), DUMMY_ENT, jnp.int32)])
    rel = jnp.concatenate([rel_idx, jnp.full((npad,), DUMMY_REL, jnp.int32)])
    return ent, rel


@jax.jit
def kernel(ent_emb_sr, ent_emb_tg, rel_emb_sr, rel_emb_tg, u_mul_s_sr, vt_sr,
           u_mul_s_tg, vt_tg, W_proj, b_proj, rel_ent_idx_sr, rel_rel_idx_sr,
           rel_ent_idx_tg, rel_rel_idx_tg):
    b2d = b_proj.reshape(1, D)
    zeros32 = jnp.zeros((32, D), jnp.float32)
    ent_sr, rel_sr = _pad_edges(rel_ent_idx_sr, rel_rel_idx_sr)
    ent_tg, rel_tg = _pad_edges(rel_ent_idx_tg, rel_rel_idx_tg)
    table2 = jnp.stack([_aug_table(rel_emb_sr), _aug_table(rel_emb_tg)])
    agg2, rinv2 = _sc_agg(table2, jnp.stack([ent_sr, ent_tg]),
                          jnp.stack([rel_sr, rel_tg]), zeros32)
    rinv2 = rinv2.reshape(2, NE_PAD, D)
    outs = []
    for sd, (e0, u, vt) in enumerate([
            (ent_emb_sr, u_mul_s_sr, vt_sr),
            (ent_emb_tg, u_mul_s_tg, vt_tg)]):
        y1, s = _stage_a(vt, e0, u)
        outs.append(_stage_b(y1, s, W_proj, b2d, e0, u, sd, agg2, rinv2))
    return jnp.concatenate(outs, axis=0)


# speedup vs baseline: 1.0889x; 1.0889x over previous
import functools
import jax
import jax.numpy as jnp
from jax import lax
from jax.experimental import pallas as pl
from jax.experimental.pallas import tpu as pltpu
from jax.experimental.pallas import tpu_sc as plsc

D = 128
N_ENT = 50000
BLK = 2000
NBLK = N_ENT // BLK

NSUB = 16
LANES = 16
NNZ_PAD = 204800
SLICE = NNZ_PAD // NSUB
EB = 1280
NBLK_E = SLICE // EB
NSTEP_E = EB // LANES
NE_PAD = 65536
CCH = 8192
NCHUNK_SC = 4
ROWS_PT = CCH // NSUB
TROWS = 1024
DUMMY_REL = 1012
DUMMY_ENT = NE_PAD - 1


_NFULL = N_ENT // D
_TAIL = N_ENT - _NFULL * D


def _ka_body(vt_ref, x_ref, out_ref):
    def step(j, acc):
        a = vt_ref[:, pl.ds(j * D, D)]
        b = x_ref[pl.ds(j * D, D), :]
        return acc + jnp.dot(a, b, preferred_element_type=jnp.float32)

    acc = jax.lax.fori_loop(0, _NFULL, step,
                            jnp.zeros((D, D), jnp.float32))
    a = vt_ref[:, pl.ds(_NFULL * D, _TAIL)]
    b = x_ref[pl.ds(_NFULL * D, _TAIL), :]
    out_ref[...] = acc + jnp.dot(a, b, preferred_element_type=jnp.float32)


def _mm_vt(vt, x):
    return pl.pallas_call(
        _ka_body,
        grid=(1,),
        in_specs=[
            pl.BlockSpec((D, N_ENT), lambda k: (0, 0)),
            pl.BlockSpec((N_ENT, D), lambda k: (0, 0)),
        ],
        out_specs=pl.BlockSpec((D, D), lambda k: (0, 0)),
        out_shape=jax.ShapeDtypeStruct((D, D), jnp.float32),
    )(vt, x)


def _stage_a(vt, e0, u):
    return _mm_vt(vt, e0), _mm_vt(vt, u)


def _kb_body(y1_ref, s_ref, wp_ref, b_ref, e0_ref, u_ref, agg_ref, rinv_ref,
             out_ref):
    y1 = y1_ref[...]
    y2 = y1 + jnp.dot(s_ref[...], y1, preferred_element_type=jnp.float32)
    ysum = y1 + y2
    e = e0_ref[...] + jnp.dot(u_ref[...], ysum,
                              preferred_element_type=jnp.float32)
    rel = agg_ref[0] * rinv_ref[0]
    acc = jnp.dot(e, wp_ref[0:D, :], preferred_element_type=jnp.float32)
    acc += jnp.dot(rel, wp_ref[D:2 * D, :], preferred_element_type=jnp.float32)
    out_ref[...] = jnp.maximum(acc + b_ref[...], 0.0)


def _stage_b(y1, s, w_proj, b2d, e0, u, sd, agg2, rinv2):
    return pl.pallas_call(
        _kb_body,
        grid=(NBLK,),
        in_specs=[
            pl.BlockSpec((D, D), lambda k: (0, 0)),
            pl.BlockSpec((D, D), lambda k: (0, 0)),
            pl.BlockSpec((2 * D, D), lambda k: (0, 0)),
            pl.BlockSpec((1, D), lambda k: (0, 0)),
            pl.BlockSpec((BLK, D), lambda k: (k, 0)),
            pl.BlockSpec((BLK, D), lambda k: (k, 0)),
            pl.BlockSpec((1, BLK, D), lambda k, sd=sd: (sd, k, 0)),
            pl.BlockSpec((1, BLK, D), lambda k, sd=sd: (sd, k, 0)),
        ],
        out_specs=pl.BlockSpec((BLK, D), lambda k: (k, 0)),
        out_shape=jax.ShapeDtypeStruct((N_ENT, D), jnp.float32),
        compiler_params=pltpu.CompilerParams(
            dimension_semantics=("arbitrary",),
        ),
    )(y1, s, w_proj, b2d, e0, u, agg2, rinv2)


def _make_sc_body(ns):
    def _sc_body(table_h, ent_h, rel_h, zeros_h, agg_h, rinv_h, ebuf, rbuf,
                 lbuf, rbufc, ring, zb, cgrid, cexp1, idx64, acc, acc_c,
                 sems, sems2):
        c = lax.axis_index("c")
        s = lax.axis_index("s")
        iota16 = lax.iota(jnp.int32, LANES)
        z16f = jnp.zeros((LANES,), jnp.float32)
        ones16 = jnp.ones((LANES,), jnp.float32)
        pltpu.sync_copy(zeros_h, zb)
        for q in range(4):
            idx64[pl.ds(q * LANES, LANES)] = iota16 + q * LANES

        def side_body(sd, _):
            def fire_edges(j):
                off = s * SLICE + j * EB
                buf = j % 2
                pltpu.async_copy(ent_h.at[sd, pl.ds(off, EB)], ebuf.at[buf],
                                 sems2.at[buf])
                pltpu.async_copy(rel_h.at[sd, pl.ds(off, EB)], rbuf.at[buf],
                                 sems2.at[buf])

            def wait_edges(j):
                off = s * SLICE + j * EB
                buf = j % 2
                pltpu.make_async_copy(ent_h.at[sd, pl.ds(off, EB)],
                                      ebuf.at[buf], sems2.at[buf]).wait()
                pltpu.make_async_copy(rel_h.at[sd, pl.ds(off, EB)],
                                      rbuf.at[buf], sems2.at[buf]).wait()

            def fire_gather(k):
                pltpu.async_copy(table_h.at[sd].at[rbufc.at[k & 3]],
                                 ring.at[k & 1], sems.at[k & 1])

            def scatter_batch(k):
                pltpu.make_async_copy(table_h.at[sd].at[rbufc.at[k & 3]],
                                      ring.at[k & 1], sems.at[k & 1]).wait()
                pltpu.sync_copy(ring.at[k & 1], acc.at[lbuf.at[k & 3]],
                                add=True)

            def chunk_body(p, _):
                lo = (2 * p + c) * CCH

                def zacc(q, _):
                    pltpu.sync_copy(zb,
                                    acc.at[pl.ds(s * ROWS_PT + q * 32, 32)])
                    return 0

                lax.fori_loop(0, ROWS_PT // 32, zacc, 0)
                pltpu.sync_copy(zb.at[pl.ds(0, 4)], acc_c.at[pl.ds(4 * s, 4)])

                def zg(r, _):
                    for q in range(8):
                        cgrid[r, pl.ds(q * LANES, LANES)] = z16f
                    return 0

                lax.fori_loop(0, 64, zg, 0)
                plsc.subcore_barrier()

                fire_edges(0)

                def blk(j, cur):
                    wait_edges(j)

                    @pl.when(j + 1 < NBLK_E)
                    def _():
                        fire_edges(j + 1)

                    def step(i, cur):
                        e = ebuf[j % 2, pl.ds(i * LANES, LANES)]
                        r = rbuf[j % 2, pl.ds(i * LANES, LANES)]
                        m = (e >= lo) & (e < lo + CCH)
                        mi = m.astype(jnp.int32)
                        el = e - lo
                        incl = plsc.cumsum(mi)
                        pos = (incl - mi) + cur
                        plsc.store_scatter(lbuf, [(pos >> 7) & 3, pos & 127],
                                           el, mask=m)
                        plsc.store_scatter(rbufc, [(pos >> 7) & 3, pos & 127],
                                           r, mask=m)
                        plsc.addupdate_scatter(cgrid, [el >> 7, el & 127],
                                               ones16, mask=m)
                        new = cur + incl[15]
                        kx = new >> 7

                        @pl.when(kx > (cur >> 7))
                        def _():
                            @pl.when(kx >= 2)
                            def _():
                                scatter_batch(kx - 2)

                            fire_gather(kx - 1)

                        return new

                    return lax.fori_loop(0, NSTEP_E, step, cur)

                cur = lax.fori_loop(0, NBLK_E, blk, jnp.int32(0))

                for q in range(8):
                    pos = iota16 + (cur + q * LANES)
                    plsc.store_scatter(lbuf, [(pos >> 7) & 3, pos & 127],
                                       jnp.zeros((LANES,), jnp.int32))
                    plsc.store_scatter(rbufc, [(pos >> 7) & 3, pos & 127],
                                       jnp.full((LANES,), DUMMY_REL,
                                                jnp.int32))
                kf = cur >> 7

                @pl.when(kf >= 1)
                def _():
                    scatter_batch(kf - 1)

                fire_gather(kf)
                scatter_batch(kf)
                pltpu.sync_copy(cgrid, acc_c.at[idx64], add=True)
                plsc.subcore_barrier()

                pltpu.sync_copy(acc.at[pl.ds(s * ROWS_PT, ROWS_PT)],
                                agg_h.at[sd, pl.ds(lo + s * ROWS_PT,
                                                   ROWS_PT)])
                pltpu.sync_copy(acc_c, cgrid)

                def grp(g, _):
                    def ex(q, _):
                        el0 = s * ROWS_PT + g * 32 + q * LANES
                        cv = cgrid[el0 >> 7, pl.ds(el0 & 127, LANES)]
                        inv = 1.0 / (cv + 1e-05)
                        for l in range(LANES):
                            v = jnp.full((LANES,), inv[l], jnp.float32)
                            for k in range(8):
                                cexp1[pl.ds((q * LANES + l) * D + k * LANES,
                                            LANES)] = v
                        return 0

                    lax.fori_loop(0, 2, ex, 0)
                    base = (lo + s * ROWS_PT + g * 32) * D
                    pltpu.sync_copy(cexp1.at[pl.ds(0, 32 * D)],
                                    rinv_h.at[sd, pl.ds(base, 32 * D)])
                    return 0

                lax.fori_loop(0, ROWS_PT // 32, grp, 0)
                plsc.subcore_barrier()
                return 0

            lax.fori_loop(0, NCHUNK_SC, chunk_body, 0)
            return 0

        lax.fori_loop(0, ns, side_body, 0)

    return _sc_body


def _sc_agg(table2, ent2, rel2, zeros32):
    ns = table2.shape[0]
    run = functools.partial(
        pl.kernel,
        out_type=[
            jax.ShapeDtypeStruct((ns, NE_PAD, D), jnp.float32),
            jax.ShapeDtypeStruct((ns, NE_PAD * D), jnp.float32),
        ],
        mesh=plsc.VectorSubcoreMesh(core_axis_name="c", subcore_axis_name="s"),
        compiler_params=pltpu.CompilerParams(needs_layout_passes=False),
        scratch_types=[
            pltpu.VMEM((2, EB), jnp.int32),
            pltpu.VMEM((2, EB), jnp.int32),
            pltpu.VMEM((4, 128), jnp.int32),
            pltpu.VMEM((4, 128), jnp.int32),
            pltpu.VMEM((2, 128, D), jnp.float32),
            pltpu.VMEM((32, D), jnp.float32),
            pltpu.VMEM((64, 128), jnp.float32),
            pltpu.VMEM((32 * D,), jnp.float32),
            pltpu.VMEM((64,), jnp.int32),
            pltpu.VMEM_SHARED((CCH, D), jnp.float32),
            pltpu.VMEM_SHARED((64, 128), jnp.float32),
            pltpu.SemaphoreType.DMA((2,)),
            pltpu.SemaphoreType.DMA((2,)),
        ],
    )(_make_sc_body(ns))
    return run(table2, ent2, rel2, zeros32)


def _aug_table(rel_emb):
    return jnp.zeros((TROWS, D), jnp.float32).at[:rel_emb.shape[0]].set(rel_emb)


def _pad_edges(ent_idx, rel_idx):
    npad = NNZ_PAD - ent_idx.shape[0]
    ent = jnp.concatenate([ent_idx, jnp.full((npad,), DUMMY_ENT, jnp.int32)])
    rel = jnp.concatenate([rel_idx, jnp.full((npad,), DUMMY_REL, jnp.int32)])
    return ent, rel


@jax.jit
def kernel(ent_emb_sr, ent_emb_tg, rel_emb_sr, rel_emb_tg, u_mul_s_sr, vt_sr,
           u_mul_s_tg, vt_tg, W_proj, b_proj, rel_ent_idx_sr, rel_rel_idx_sr,
           rel_ent_idx_tg, rel_rel_idx_tg):
    b2d = b_proj.reshape(1, D)
    zeros32 = jnp.zeros((32, D), jnp.float32)
    outs = []
    for e0, u, vt, rel_emb, eidx, ridx in [
            (ent_emb_sr, u_mul_s_sr, vt_sr, rel_emb_sr,
             rel_ent_idx_sr, rel_rel_idx_sr),
            (ent_emb_tg, u_mul_s_tg, vt_tg, rel_emb_tg,
             rel_ent_idx_tg, rel_rel_idx_tg)]:
        ent, rel = _pad_edges(eidx, ridx)
        agg1, rinv1 = _sc_agg(_aug_table(rel_emb).reshape(1, TROWS, D),
                              ent.reshape(1, NNZ_PAD),
                              rel.reshape(1, NNZ_PAD), zeros32)
        y1, s = _stage_a(vt, e0, u)
        outs.append(_stage_b(y1, s, W_proj, b2d, e0, u, 0, agg1,
                             rinv1.reshape(1, NE_PAD, D)))
    return jnp.concatenate(outs, axis=0)


# speedup vs baseline: 1.1216x; 1.0301x over previous
import functools
import jax
import jax.numpy as jnp
from jax import lax
from jax.experimental import pallas as pl
from jax.experimental.pallas import tpu as pltpu
from jax.experimental.pallas import tpu_sc as plsc

D = 128
N_ENT = 50000
BLK = 2000
NBLK = N_ENT // BLK

NSUB = 16
LANES = 16
NNZ_PAD = 204800
SLICE = NNZ_PAD // NSUB
EB = 1280
NBLK_E = SLICE // EB
NSTEP_E = EB // LANES
NE_PAD = 65536
CCH = 8192
NCHUNK_SC = 4
ROWS_PT = CCH // NSUB
TROWS = 1024
DUMMY_REL = 1012
DUMMY_ENT = NE_PAD - 1


_NFULL = N_ENT // D
_TAIL = N_ENT - _NFULL * D


def _ka_body(vt_ref, x_ref, out_ref):
    def step(j, acc):
        a = vt_ref[:, pl.ds(j * D, D)]
        b = x_ref[pl.ds(j * D, D), :]
        return acc + jnp.dot(a, b, preferred_element_type=jnp.float32)

    acc = jax.lax.fori_loop(0, _NFULL, step,
                            jnp.zeros((D, D), jnp.float32))
    a = vt_ref[:, pl.ds(_NFULL * D, _TAIL)]
    b = x_ref[pl.ds(_NFULL * D, _TAIL), :]
    out_ref[...] = acc + jnp.dot(a, b, preferred_element_type=jnp.float32)


def _mm_vt(vt, x):
    return pl.pallas_call(
        _ka_body,
        grid=(1,),
        in_specs=[
            pl.BlockSpec((D, N_ENT), lambda k: (0, 0)),
            pl.BlockSpec((N_ENT, D), lambda k: (0, 0)),
        ],
        out_specs=pl.BlockSpec((D, D), lambda k: (0, 0)),
        out_shape=jax.ShapeDtypeStruct((D, D), jnp.float32),
    )(vt, x)


def _stage_a(vt, e0, u):
    return _mm_vt(vt, e0), _mm_vt(vt, u)


def _kb_body(y1_ref, s_ref, wp_ref, b_ref, e0_ref, u_ref, agg_ref, rinv_ref,
             out_ref):
    y1 = y1_ref[...]
    y2 = y1 + jnp.dot(s_ref[...], y1, preferred_element_type=jnp.float32)
    ysum = y1 + y2
    e = e0_ref[...] + jnp.dot(u_ref[...], ysum,
                              preferred_element_type=jnp.float32)
    rel = agg_ref[0] * rinv_ref[0]
    acc = jnp.dot(e, wp_ref[0:D, :], preferred_element_type=jnp.float32)
    acc += jnp.dot(rel, wp_ref[D:2 * D, :], preferred_element_type=jnp.float32)
    out_ref[...] = jnp.maximum(acc + b_ref[...], 0.0)


def _stage_b(y1, s, w_proj, b2d, e0, u, sd, agg2, rinv2):
    return pl.pallas_call(
        _kb_body,
        grid=(NBLK,),
        in_specs=[
            pl.BlockSpec((D, D), lambda k: (0, 0)),
            pl.BlockSpec((D, D), lambda k: (0, 0)),
            pl.BlockSpec((2 * D, D), lambda k: (0, 0)),
            pl.BlockSpec((1, D), lambda k: (0, 0)),
            pl.BlockSpec((BLK, D), lambda k: (k, 0)),
            pl.BlockSpec((BLK, D), lambda k: (k, 0)),
            pl.BlockSpec((1, BLK, D), lambda k, sd=sd: (sd, k, 0)),
            pl.BlockSpec((1, BLK, D), lambda k, sd=sd: (sd, k, 0)),
        ],
        out_specs=pl.BlockSpec((BLK, D), lambda k: (k, 0)),
        out_shape=jax.ShapeDtypeStruct((N_ENT, D), jnp.float32),
        compiler_params=pltpu.CompilerParams(
            dimension_semantics=("arbitrary",),
        ),
    )(y1, s, w_proj, b2d, e0, u, agg2, rinv2)


def _make_sc_body(ns):
    def _sc_body(table_h, ent_h, rel_h, zeros_h, agg_h, rinv_h, ebuf, rbuf,
                 lbuf, rbufc, ring, zb, cgrid, cexp1, idx64, acc, acc_c,
                 sems, sems2, sems3):
        c = lax.axis_index("c")
        s = lax.axis_index("s")
        iota16 = lax.iota(jnp.int32, LANES)
        z16f = jnp.zeros((LANES,), jnp.float32)
        ones16 = jnp.ones((LANES,), jnp.float32)
        pltpu.sync_copy(zeros_h, zb)
        for q in range(4):
            idx64[pl.ds(q * LANES, LANES)] = iota16 + q * LANES

        def side_body(sd, _):
            def fire_edges(j):
                off = s * SLICE + j * EB
                buf = j % 2
                pltpu.async_copy(ent_h.at[sd, pl.ds(off, EB)], ebuf.at[buf],
                                 sems2.at[buf])
                pltpu.async_copy(rel_h.at[sd, pl.ds(off, EB)], rbuf.at[buf],
                                 sems2.at[buf])

            def wait_edges(j):
                off = s * SLICE + j * EB
                buf = j % 2
                pltpu.make_async_copy(ent_h.at[sd, pl.ds(off, EB)],
                                      ebuf.at[buf], sems2.at[buf]).wait()
                pltpu.make_async_copy(rel_h.at[sd, pl.ds(off, EB)],
                                      rbuf.at[buf], sems2.at[buf]).wait()

            def fire_gather(k):
                pltpu.async_copy(table_h.at[sd].at[rbufc.at[k & 3]],
                                 ring.at[k & 1], sems.at[k & 1])

            def wait_gather(k):
                pltpu.make_async_copy(table_h.at[sd].at[rbufc.at[k & 3]],
                                      ring.at[k & 1], sems.at[k & 1]).wait()

            def fire_scatter(k):
                pltpu.async_copy(ring.at[k & 1], acc.at[lbuf.at[k & 3]],
                                 sems3.at[k & 1], add=True)

            def wait_scatter(k):
                pltpu.make_async_copy(ring.at[k & 1], acc.at[lbuf.at[k & 3]],
                                      sems3.at[k & 1]).wait()

            def scatter_batch(k):
                wait_gather(k)
                pltpu.sync_copy(ring.at[k & 1], acc.at[lbuf.at[k & 3]],
                                add=True)

            def chunk_body(p, _):
                lo = (2 * p + c) * CCH

                def zfire(q, _):
                    pltpu.async_copy(zb,
                                     acc.at[pl.ds(s * ROWS_PT + q * 32, 32)],
                                     sems3.at[0])
                    return 0

                lax.fori_loop(0, ROWS_PT // 32, zfire, 0)
                pltpu.sync_copy(zb.at[pl.ds(0, 4)], acc_c.at[pl.ds(4 * s, 4)])

                def zdrain(q, _):
                    pltpu.make_async_copy(
                        zb, acc.at[pl.ds(s * ROWS_PT + q * 32, 32)],
                        sems3.at[0]).wait()
                    return 0

                lax.fori_loop(0, ROWS_PT // 32, zdrain, 0)

                def zg(r, _):
                    for q in range(8):
                        cgrid[r, pl.ds(q * LANES, LANES)] = z16f
                    return 0

                lax.fori_loop(0, 64, zg, 0)
                plsc.subcore_barrier()

                fire_edges(0)

                def blk(j, cur):
                    wait_edges(j)

                    @pl.when(j + 1 < NBLK_E)
                    def _():
                        fire_edges(j + 1)

                    def step(i, cur):
                        e = ebuf[j % 2, pl.ds(i * LANES, LANES)]
                        r = rbuf[j % 2, pl.ds(i * LANES, LANES)]
                        m = (e >= lo) & (e < lo + CCH)
                        mi = m.astype(jnp.int32)
                        el = e - lo
                        incl = plsc.cumsum(mi)
                        pos = (incl - mi) + cur
                        plsc.store_scatter(lbuf, [(pos >> 7) & 3, pos & 127],
                                           el, mask=m)
                        plsc.store_scatter(rbufc, [(pos >> 7) & 3, pos & 127],
                                           r, mask=m)
                        plsc.addupdate_scatter(cgrid, [el >> 7, el & 127],
                                               ones16, mask=m)
                        new = cur + incl[15]
                        kx = new >> 7

                        @pl.when(kx > (cur >> 7))
                        def _():
                            @pl.when(kx >= 3)
                            def _():
                                wait_scatter(kx - 3)

                            @pl.when(kx >= 2)
                            def _():
                                wait_gather(kx - 2)
                                fire_scatter(kx - 2)

                            fire_gather(kx - 1)

                        return new

                    return lax.fori_loop(0, NSTEP_E, step, cur)

                cur = lax.fori_loop(0, NBLK_E, blk, jnp.int32(0))

                for q in range(8):
                    pos = iota16 + (cur + q * LANES)
                    plsc.store_scatter(lbuf, [(pos >> 7) & 3, pos & 127],
                                       jnp.zeros((LANES,), jnp.int32))
                    plsc.store_scatter(rbufc, [(pos >> 7) & 3, pos & 127],
                                       jnp.full((LANES,), DUMMY_REL,
                                                jnp.int32))
                kf = cur >> 7

                @pl.when(kf >= 2)
                def _():
                    wait_scatter(kf - 2)

                @pl.when(kf >= 1)
                def _():
                    scatter_batch(kf - 1)

                fire_gather(kf)
                scatter_batch(kf)
                pltpu.sync_copy(cgrid, acc_c.at[idx64], add=True)
                plsc.subcore_barrier()

                pltpu.sync_copy(acc.at[pl.ds(s * ROWS_PT, ROWS_PT)],
                                agg_h.at[sd, pl.ds(lo + s * ROWS_PT,
                                                   ROWS_PT)])
                pltpu.sync_copy(acc_c, cgrid)

                def grp(g, _):
                    def ex(q, _):
                        el0 = s * ROWS_PT + g * 32 + q * LANES
                        cv = cgrid[el0 >> 7, pl.ds(el0 & 127, LANES)]
                        inv = 1.0 / (cv + 1e-05)
                        for l in range(LANES):
                            v = jnp.full((LANES,), inv[l], jnp.float32)
                            for k in range(8):
                                cexp1[pl.ds((q * LANES + l) * D + k * LANES,
                                            LANES)] = v
                        return 0

                    lax.fori_loop(0, 2, ex, 0)
                    base = (lo + s * ROWS_PT + g * 32) * D
                    pltpu.sync_copy(cexp1.at[pl.ds(0, 32 * D)],
                                    rinv_h.at[sd, pl.ds(base, 32 * D)])
                    return 0

                lax.fori_loop(0, ROWS_PT // 32, grp, 0)
                plsc.subcore_barrier()
                return 0

            lax.fori_loop(0, NCHUNK_SC, chunk_body, 0)
            return 0

        lax.fori_loop(0, ns, side_body, 0)

    return _sc_body


def _sc_agg(table2, ent2, rel2, zeros32):
    ns = table2.shape[0]
    run = functools.partial(
        pl.kernel,
        out_type=[
            jax.ShapeDtypeStruct((ns, NE_PAD, D), jnp.float32),
            jax.ShapeDtypeStruct((ns, NE_PAD * D), jnp.float32),
        ],
        mesh=plsc.VectorSubcoreMesh(core_axis_name="c", subcore_axis_name="s"),
        compiler_params=pltpu.CompilerParams(needs_layout_passes=False),
        scratch_types=[
            pltpu.VMEM((2, EB), jnp.int32),
            pltpu.VMEM((2, EB), jnp.int32),
            pltpu.VMEM((4, 128), jnp.int32),
            pltpu.VMEM((4, 128), jnp.int32),
            pltpu.VMEM((2, 128, D), jnp.float32),
            pltpu.VMEM((32, D), jnp.float32),
            pltpu.VMEM((64, 128), jnp.float32),
            pltpu.VMEM((32 * D,), jnp.float32),
            pltpu.VMEM((64,), jnp.int32),
            pltpu.VMEM_SHARED((CCH, D), jnp.float32),
            pltpu.VMEM_SHARED((64, 128), jnp.float32),
            pltpu.SemaphoreType.DMA((2,)),
            pltpu.SemaphoreType.DMA((2,)),
            pltpu.SemaphoreType.DMA((2,)),
        ],
    )(_make_sc_body(ns))
    return run(table2, ent2, rel2, zeros32)


def _aug_table(rel_emb):
    return jnp.zeros((TROWS, D), jnp.float32).at[:rel_emb.shape[0]].set(rel_emb)


def _pad_edges(ent_idx, rel_idx):
    npad = NNZ_PAD - ent_idx.shape[0]
    ent = jnp.concatenate([ent_idx, jnp.full((npad,), DUMMY_ENT, jnp.int32)])
    rel = jnp.concatenate([rel_idx, jnp.full((npad,), DUMMY_REL, jnp.int32)])
    return ent, rel


@jax.jit
def kernel(ent_emb_sr, ent_emb_tg, rel_emb_sr, rel_emb_tg, u_mul_s_sr, vt_sr,
           u_mul_s_tg, vt_tg, W_proj, b_proj, rel_ent_idx_sr, rel_rel_idx_sr,
           rel_ent_idx_tg, rel_rel_idx_tg):
    b2d = b_proj.reshape(1, D)
    zeros32 = jnp.zeros((32, D), jnp.float32)
    outs = []
    for e0, u, vt, rel_emb, eidx, ridx in [
            (ent_emb_sr, u_mul_s_sr, vt_sr, rel_emb_sr,
             rel_ent_idx_sr, rel_rel_idx_sr),
            (ent_emb_tg, u_mul_s_tg, vt_tg, rel_emb_tg,
             rel_ent_idx_tg, rel_rel_idx_tg)]:
        ent, rel = _pad_edges(eidx, ridx)
        agg1, rinv1 = _sc_agg(_aug_table(rel_emb).reshape(1, TROWS, D),
                              ent.reshape(1, NNZ_PAD),
                              rel.reshape(1, NNZ_PAD), zeros32)
        y1, s = _stage_a(vt, e0, u)
        outs.append(_stage_b(y1, s, W_proj, b2d, e0, u, 0, agg1,
                             rinv1.reshape(1, NE_PAD, D)))
    return jnp.concatenate(outs, axis=0)


# speedup vs baseline: 3.6866x; 3.2868x over previous
import functools
import jax
import jax.numpy as jnp
from jax import lax
from jax.experimental import pallas as pl
from jax.experimental.pallas import tpu as pltpu
from jax.experimental.pallas import tpu_sc as plsc

D = 128
N_ENT = 50000
BLK = 2000
NBLK = N_ENT // BLK

NSUB = 16
LANES = 16
NNZ_PAD = 204800
SLICE = NNZ_PAD // NSUB
EB = 1280
NBLK_E = SLICE // EB
NSTEP_E = EB // LANES
NE_PAD = 65536
CCH = 8192
NCHUNK_SC = 4
ROWS_PT = CCH // NSUB
TROWS = 1024
DUMMY_REL = 1012
DUMMY_ENT = NE_PAD - 1


_NFULL = N_ENT // D
_TAIL = N_ENT - _NFULL * D


def _ka_body(vt_ref, x_ref, out_ref):
    def step(j, acc):
        a = vt_ref[:, pl.ds(j * D, D)]
        b = x_ref[pl.ds(j * D, D), :]
        return acc + jnp.dot(a, b, preferred_element_type=jnp.float32)

    acc = jax.lax.fori_loop(0, _NFULL, step,
                            jnp.zeros((D, D), jnp.float32))
    a = vt_ref[:, pl.ds(_NFULL * D, _TAIL)]
    b = x_ref[pl.ds(_NFULL * D, _TAIL), :]
    out_ref[...] = acc + jnp.dot(a, b, preferred_element_type=jnp.float32)


def _mm_vt(vt, x):
    return pl.pallas_call(
        _ka_body,
        grid=(1,),
        in_specs=[
            pl.BlockSpec((D, N_ENT), lambda k: (0, 0)),
            pl.BlockSpec((N_ENT, D), lambda k: (0, 0)),
        ],
        out_specs=pl.BlockSpec((D, D), lambda k: (0, 0)),
        out_shape=jax.ShapeDtypeStruct((D, D), jnp.float32),
    )(vt, x)


def _stage_a(vt, e0, u):
    return _mm_vt(vt, e0), _mm_vt(vt, u)


def _kb_body(y1_ref, s_ref, wp_ref, b_ref, e0_ref, u_ref, agg_ref, rinv_ref,
             out_ref):
    y1 = y1_ref[...]
    y2 = y1 + jnp.dot(s_ref[...], y1, preferred_element_type=jnp.float32)
    ysum = y1 + y2
    e = e0_ref[...] + jnp.dot(u_ref[...], ysum,
                              preferred_element_type=jnp.float32)
    rel = agg_ref[0] * rinv_ref[0]
    acc = jnp.dot(e, wp_ref[0:D, :], preferred_element_type=jnp.float32)
    acc += jnp.dot(rel, wp_ref[D:2 * D, :], preferred_element_type=jnp.float32)
    out_ref[...] = jnp.maximum(acc + b_ref[...], 0.0)


def _stage_b(y1, s, w_proj, b2d, e0, u, sd, agg2, rinv2):
    return pl.pallas_call(
        _kb_body,
        grid=(NBLK,),
        in_specs=[
            pl.BlockSpec((D, D), lambda k: (0, 0)),
            pl.BlockSpec((D, D), lambda k: (0, 0)),
            pl.BlockSpec((2 * D, D), lambda k: (0, 0)),
            pl.BlockSpec((1, D), lambda k: (0, 0)),
            pl.BlockSpec((BLK, D), lambda k: (k, 0)),
            pl.BlockSpec((BLK, D), lambda k: (k, 0)),
            pl.BlockSpec((1, BLK, D), lambda k, sd=sd: (sd, k, 0)),
            pl.BlockSpec((1, BLK, D), lambda k, sd=sd: (sd, k, 0)),
        ],
        out_specs=pl.BlockSpec((BLK, D), lambda k: (k, 0)),
        out_shape=jax.ShapeDtypeStruct((N_ENT, D), jnp.float32),
        compiler_params=pltpu.CompilerParams(
            dimension_semantics=("arbitrary",),
        ),
    )(y1, s, w_proj, b2d, e0, u, agg2, rinv2)


def _make_sc_body(ns):
    def _sc_body(table_h, ent_h, rel_h, zeros_h, agg_h, rinv_h, ebuf, rbuf,
                 lbuf, rbufc, ring, zb, cgrid, cexp1, idx64, acc, tbl_s,
                 acc_c, sems, sems2, sems3):
        c = lax.axis_index("c")
        s = lax.axis_index("s")
        iota16 = lax.iota(jnp.int32, LANES)
        z16f = jnp.zeros((LANES,), jnp.float32)
        ones16 = jnp.ones((LANES,), jnp.float32)
        pltpu.sync_copy(zeros_h, zb)
        for q in range(4):
            idx64[pl.ds(q * LANES, LANES)] = iota16 + q * LANES

        def side_body(sd, _):
            pltpu.sync_copy(table_h.at[sd, pl.ds(64 * s, 64)],
                            tbl_s.at[pl.ds(64 * s, 64)])

            def fire_edges(j):
                off = s * SLICE + j * EB
                buf = j % 2
                pltpu.async_copy(ent_h.at[sd, pl.ds(off, EB)], ebuf.at[buf],
                                 sems2.at[buf])
                pltpu.async_copy(rel_h.at[sd, pl.ds(off, EB)], rbuf.at[buf],
                                 sems2.at[buf])

            def wait_edges(j):
                off = s * SLICE + j * EB
                buf = j % 2
                pltpu.make_async_copy(ent_h.at[sd, pl.ds(off, EB)],
                                      ebuf.at[buf], sems2.at[buf]).wait()
                pltpu.make_async_copy(rel_h.at[sd, pl.ds(off, EB)],
                                      rbuf.at[buf], sems2.at[buf]).wait()

            def fire_gather(k):
                pltpu.async_copy(tbl_s.at[rbufc.at[k & 3]],
                                 ring.at[k & 1], sems.at[k & 1])

            def wait_gather(k):
                pltpu.make_async_copy(tbl_s.at[rbufc.at[k & 3]],
                                      ring.at[k & 1], sems.at[k & 1]).wait()

            def fire_scatter(k):
                pltpu.async_copy(ring.at[k & 1], acc.at[lbuf.at[k & 3]],
                                 sems3.at[k & 1], add=True)

            def wait_scatter(k):
                pltpu.make_async_copy(ring.at[k & 1], acc.at[lbuf.at[k & 3]],
                                      sems3.at[k & 1]).wait()

            def scatter_batch(k):
                wait_gather(k)
                pltpu.sync_copy(ring.at[k & 1], acc.at[lbuf.at[k & 3]],
                                add=True)

            def chunk_body(p, _):
                lo = (2 * p + c) * CCH

                def zfire(q, _):
                    pltpu.async_copy(zb,
                                     acc.at[pl.ds(s * ROWS_PT + q * 32, 32)],
                                     sems3.at[0])
                    return 0

                lax.fori_loop(0, ROWS_PT // 32, zfire, 0)
                pltpu.sync_copy(zb.at[pl.ds(0, 4)], acc_c.at[pl.ds(4 * s, 4)])

                def zdrain(q, _):
                    pltpu.make_async_copy(
                        zb, acc.at[pl.ds(s * ROWS_PT + q * 32, 32)],
                        sems3.at[0]).wait()
                    return 0

                lax.fori_loop(0, ROWS_PT // 32, zdrain, 0)

                def zg(r, _):
                    for q in range(8):
                        cgrid[r, pl.ds(q * LANES, LANES)] = z16f
                    return 0

                lax.fori_loop(0, 64, zg, 0)
                plsc.subcore_barrier()

                fire_edges(0)

                def blk(j, cur):
                    wait_edges(j)

                    @pl.when(j + 1 < NBLK_E)
                    def _():
                        fire_edges(j + 1)

                    def step(i, cur):
                        e = ebuf[j % 2, pl.ds(i * LANES, LANES)]
                        r = rbuf[j % 2, pl.ds(i * LANES, LANES)]
                        m = (e >= lo) & (e < lo + CCH)
                        mi = m.astype(jnp.int32)
                        el = e - lo
                        incl = plsc.cumsum(mi)
                        pos = (incl - mi) + cur
                        plsc.store_scatter(lbuf, [(pos >> 7) & 3, pos & 127],
                                           el, mask=m)
                        plsc.store_scatter(rbufc, [(pos >> 7) & 3, pos & 127],
                                           r, mask=m)
                        plsc.addupdate_scatter(cgrid, [el >> 7, el & 127],
                                               ones16, mask=m)
                        new = cur + incl[15]
                        kx = new >> 7

                        @pl.when(kx > (cur >> 7))
                        def _():
                            @pl.when(kx >= 3)
                            def _():
                                wait_scatter(kx - 3)

                            @pl.when(kx >= 2)
                            def _():
                                wait_gather(kx - 2)
                                fire_scatter(kx - 2)

                            fire_gather(kx - 1)

                        return new

                    return lax.fori_loop(0, NSTEP_E, step, cur)

                cur = lax.fori_loop(0, NBLK_E, blk, jnp.int32(0))

                for q in range(8):
                    pos = iota16 + (cur + q * LANES)
                    plsc.store_scatter(lbuf, [(pos >> 7) & 3, pos & 127],
                                       jnp.zeros((LANES,), jnp.int32))
                    plsc.store_scatter(rbufc, [(pos >> 7) & 3, pos & 127],
                                       jnp.full((LANES,), DUMMY_REL,
                                                jnp.int32))
                kf = cur >> 7

                @pl.when(kf >= 2)
                def _():
                    wait_scatter(kf - 2)

                @pl.when(kf >= 1)
                def _():
                    scatter_batch(kf - 1)

                fire_gather(kf)
                scatter_batch(kf)
                pltpu.sync_copy(cgrid, acc_c.at[idx64], add=True)
                plsc.subcore_barrier()

                pltpu.sync_copy(acc.at[pl.ds(s * ROWS_PT, ROWS_PT)],
                                agg_h.at[sd, pl.ds(lo + s * ROWS_PT,
                                                   ROWS_PT)])
                pltpu.sync_copy(acc_c, cgrid)

                def grp(g, _):
                    def ex(q, _):
                        el0 = s * ROWS_PT + g * 32 + q * LANES
                        cv = cgrid[el0 >> 7, pl.ds(el0 & 127, LANES)]
                        inv = 1.0 / (cv + 1e-05)
                        for l in range(LANES):
                            v = jnp.full((LANES,), inv[l], jnp.float32)
                            for k in range(8):
                                cexp1[pl.ds((q * LANES + l) * D + k * LANES,
                                            LANES)] = v
                        return 0

                    lax.fori_loop(0, 2, ex, 0)
                    base = (lo + s * ROWS_PT + g * 32) * D
                    pltpu.sync_copy(cexp1.at[pl.ds(0, 32 * D)],
                                    rinv_h.at[sd, pl.ds(base, 32 * D)])
                    return 0

                lax.fori_loop(0, ROWS_PT // 32, grp, 0)
                plsc.subcore_barrier()
                return 0

            lax.fori_loop(0, NCHUNK_SC, chunk_body, 0)
            return 0

        lax.fori_loop(0, ns, side_body, 0)

    return _sc_body


def _sc_agg(table2, ent2, rel2, zeros32):
    ns = table2.shape[0]
    run = functools.partial(
        pl.kernel,
        out_type=[
            jax.ShapeDtypeStruct((ns, NE_PAD, D), jnp.float32),
            jax.ShapeDtypeStruct((ns, NE_PAD * D), jnp.float32),
        ],
        mesh=plsc.VectorSubcoreMesh(core_axis_name="c", subcore_axis_name="s"),
        compiler_params=pltpu.CompilerParams(needs_layout_passes=False),
        scratch_types=[
            pltpu.VMEM((2, EB), jnp.int32),
            pltpu.VMEM((2, EB), jnp.int32),
            pltpu.VMEM((4, 128), jnp.int32),
            pltpu.VMEM((4, 128), jnp.int32),
            pltpu.VMEM((2, 128, D), jnp.float32),
            pltpu.VMEM((32, D), jnp.float32),
            pltpu.VMEM((64, 128), jnp.float32),
            pltpu.VMEM((32 * D,), jnp.float32),
            pltpu.VMEM((64,), jnp.int32),
            pltpu.VMEM_SHARED((CCH, D), jnp.float32),
            pltpu.VMEM_SHARED((TROWS, D), jnp.float32),
            pltpu.VMEM_SHARED((64, 128), jnp.float32),
            pltpu.SemaphoreType.DMA((2,)),
            pltpu.SemaphoreType.DMA((2,)),
            pltpu.SemaphoreType.DMA((2,)),
        ],
    )(_make_sc_body(ns))
    return run(table2, ent2, rel2, zeros32)


def _aug_table(rel_emb):
    return jnp.zeros((TROWS, D), jnp.float32).at[:rel_emb.shape[0]].set(rel_emb)


def _pad_edges(ent_idx, rel_idx):
    npad = NNZ_PAD - ent_idx.shape[0]
    ent = jnp.concatenate([ent_idx, jnp.full((npad,), DUMMY_ENT, jnp.int32)])
    rel = jnp.concatenate([rel_idx, jnp.full((npad,), DUMMY_REL, jnp.int32)])
    return ent, rel


@jax.jit
def kernel(ent_emb_sr, ent_emb_tg, rel_emb_sr, rel_emb_tg, u_mul_s_sr, vt_sr,
           u_mul_s_tg, vt_tg, W_proj, b_proj, rel_ent_idx_sr, rel_rel_idx_sr,
           rel_ent_idx_tg, rel_rel_idx_tg):
    b2d = b_proj.reshape(1, D)
    zeros32 = jnp.zeros((32, D), jnp.float32)
    outs = []
    for e0, u, vt, rel_emb, eidx, ridx in [
            (ent_emb_sr, u_mul_s_sr, vt_sr, rel_emb_sr,
             rel_ent_idx_sr, rel_rel_idx_sr),
            (ent_emb_tg, u_mul_s_tg, vt_tg, rel_emb_tg,
             rel_ent_idx_tg, rel_rel_idx_tg)]:
        ent, rel = _pad_edges(eidx, ridx)
        agg1, rinv1 = _sc_agg(_aug_table(rel_emb).reshape(1, TROWS, D),
                              ent.reshape(1, NNZ_PAD),
                              rel.reshape(1, NNZ_PAD), zeros32)
        y1, s = _stage_a(vt, e0, u)
        outs.append(_stage_b(y1, s, W_proj, b2d, e0, u, 0, agg1,
                             rinv1.reshape(1, NE_PAD, D)))
    return jnp.concatenate(outs, axis=0)
